# baseline (speedup 1.0000x reference)
import functools

import jax
import jax.numpy as jnp
from jax import lax
from jax.experimental import pallas as pl
from jax.experimental.pallas import tpu as pltpu
from jax.experimental.pallas import tpu_sc as plsc

VOCAB = 1000
VPAD = 1024
BATCH = 16384
SEQ = 200
DIM = 64

NC = 2
NS = 16
NW = NC * NS
LANES = 16
ROWS_PER_TILE = BATCH // NW
GROUPS_PER_TILE = ROWS_PER_TILE // LANES


def _proj_kernel(t_ref, wt_ref, b_ref, o_ref):
    o_ref[...] = (
        jnp.dot(t_ref[...], wt_ref[...], preferred_element_type=jnp.float32)
        + b_ref[...]
    )


def _sc_body(v_hbm, x_hbm, out_hbm, v_v, x_v, out_v):
    wid = lax.axis_index("s") * NC + lax.axis_index("c")
    row0 = wid * ROWS_PER_TILE

    pltpu.sync_copy(v_hbm, v_v)

    lane = lax.iota(jnp.int32, 16)

    UNROLL = 8
    CHUNK = 256
    GROUPS_PER_CHUNK = CHUNK // LANES

    def chunk_body(c, _):
        pltpu.sync_copy(x_hbm.at[pl.ds(row0 + c * CHUNK, CHUNK)], x_v)

        def group_body(g, _):
            rows = lane + g * LANES

            def tok_body(i, accs):
                accs = list(accs)
                t0 = i * UNROLL
                for j in range(UNROLL):
                    cols = jnp.full((16,), t0 + j, jnp.int32)
                    idxs = plsc.load_gather(x_v, [rows, cols])
                    vals = plsc.load_gather(v_v, [idxs])
                    accs[j % 4] = accs[j % 4] + vals
                return tuple(accs)

            zero = jnp.zeros((16,), jnp.float32)
            a = lax.fori_loop(0, SEQ // UNROLL, tok_body,
                              (zero, zero, zero, zero))
            acc = (a[0] + a[1]) + (a[2] + a[3])
            z = acc * (1.0 / SEQ)
            y = 1.0 / (1.0 + jnp.exp(-z))
            out_v[pl.ds(c * CHUNK + g * LANES, LANES)] = y
            return 0

        lax.fori_loop(0, GROUPS_PER_CHUNK, group_body, 0)
        return 0

    lax.fori_loop(0, ROWS_PER_TILE // CHUNK, chunk_body, 0)

    pltpu.sync_copy(out_v, out_hbm.at[pl.ds(row0, ROWS_PER_TILE)])


_sc_call = functools.partial(
    pl.kernel,
    out_type=jax.ShapeDtypeStruct((BATCH,), jnp.float32),
    mesh=plsc.VectorSubcoreMesh(core_axis_name="c", subcore_axis_name="s"),
    scratch_types=[
        pltpu.VMEM((VPAD,), jnp.float32),
        pltpu.VMEM((256, SEQ), jnp.int32),
        pltpu.VMEM((ROWS_PER_TILE,), jnp.float32),
    ],
    compiler_params=pltpu.CompilerParams(
        needs_layout_passes=False, use_tc_tiling_on_sc=True
    ),
)(_sc_body)


def kernel(x, emb_table, W, b):
    table_pad = jnp.pad(emb_table, ((0, VPAD - VOCAB), (0, 0)))
    wt = W.reshape(DIM, 1)
    b2 = b.reshape(1, 1)
    v = pl.pallas_call(
        _proj_kernel,
        out_shape=jax.ShapeDtypeStruct((VPAD, 1), jnp.float32),
    )(table_pad, wt, b2)
    out = _sc_call(v.reshape(VPAD), x)
    return out.reshape(BATCH, 1)


# speedup vs baseline: 1.1521x; 1.1521x over previous
import functools

import jax
import jax.numpy as jnp
from jax import lax
from jax.experimental import pallas as pl
from jax.experimental.pallas import tpu as pltpu
from jax.experimental.pallas import tpu_sc as plsc

VOCAB = 1000
VPAD = 1024
BATCH = 16384
SEQ = 200
DIM = 64

NC = 2
NS = 16
NW = NC * NS
LANES = 16
ROWS_PER_TILE = BATCH // NW
GROUPS_PER_TILE = ROWS_PER_TILE // LANES


def _proj_kernel(t_ref, wt_ref, b_ref, o_ref):
    o_ref[...] = (
        jnp.dot(t_ref[...], wt_ref[...], preferred_element_type=jnp.float32)
        + b_ref[...]
    )


def _sc_body(v_hbm, x_hbm, out_hbm, v_v, x_v, out_v, xf_v):
    wid = lax.axis_index("s") * NC + lax.axis_index("c")
    row0 = wid * ROWS_PER_TILE

    pltpu.sync_copy(v_hbm, v_v)

    lane = lax.iota(jnp.int32, 16)

    UNROLL = 8
    CHUNK = 256
    GROUPS_PER_CHUNK = CHUNK // LANES

    lane_off = lane * SEQ

    def chunk_body(c, _):
        pltpu.sync_copy(x_hbm.at[pl.ds(row0 + c * CHUNK, CHUNK)], x_v)

        def group_body(g, _):
            for r in range(LANES):
                row = g * LANES + r
                for t in range(12):
                    xf_v[pl.ds(r * SEQ + t * 16, 16)] = x_v[row, pl.ds(t * 16, 16)]
                xf_v[pl.ds(r * SEQ + SEQ - 16, 16)] = x_v[row, pl.ds(SEQ - 16, 16)]

            def tok_body(i, accs):
                accs = list(accs)
                t0 = i * UNROLL
                for j in range(UNROLL):
                    offs = lane_off + (t0 + j)
                    idxs = plsc.load_gather(xf_v, [offs])
                    vals = plsc.load_gather(v_v, [idxs])
                    accs[j % 4] = accs[j % 4] + vals
                return tuple(accs)

            zero = jnp.zeros((16,), jnp.float32)
            a = lax.fori_loop(0, SEQ // UNROLL, tok_body,
                              (zero, zero, zero, zero))
            acc = (a[0] + a[1]) + (a[2] + a[3])
            z = acc * (1.0 / SEQ)
            y = 1.0 / (1.0 + jnp.exp(-z))
            out_v[pl.ds(c * CHUNK + g * LANES, LANES)] = y
            return 0

        lax.fori_loop(0, GROUPS_PER_CHUNK, group_body, 0)
        return 0

    lax.fori_loop(0, ROWS_PER_TILE // CHUNK, chunk_body, 0)

    pltpu.sync_copy(out_v, out_hbm.at[pl.ds(row0, ROWS_PER_TILE)])


_sc_call = functools.partial(
    pl.kernel,
    out_type=jax.ShapeDtypeStruct((BATCH,), jnp.float32),
    mesh=plsc.VectorSubcoreMesh(core_axis_name="c", subcore_axis_name="s"),
    scratch_types=[
        pltpu.VMEM((VPAD,), jnp.float32),
        pltpu.VMEM((256, SEQ), jnp.int32),
        pltpu.VMEM((ROWS_PER_TILE,), jnp.float32),
        pltpu.VMEM((LANES * SEQ,), jnp.int32),
    ],
    compiler_params=pltpu.CompilerParams(
        needs_layout_passes=False, use_tc_tiling_on_sc=True
    ),
)(_sc_body)


def kernel(x, emb_table, W, b):
    table_pad = jnp.pad(emb_table, ((0, VPAD - VOCAB), (0, 0)))
    wt = W.reshape(DIM, 1)
    b2 = b.reshape(1, 1)
    v = pl.pallas_call(
        _proj_kernel,
        out_shape=jax.ShapeDtypeStruct((VPAD, 1), jnp.float32),
    )(table_pad, wt, b2)
    out = _sc_call(v.reshape(VPAD), x)
    return out.reshape(BATCH, 1)


# speedup vs baseline: 2.0888x; 1.8130x over previous
import functools

import jax
import jax.numpy as jnp
from jax import lax
from jax.experimental import pallas as pl
from jax.experimental.pallas import tpu as pltpu
from jax.experimental.pallas import tpu_sc as plsc

VOCAB = 1000
VPAD = 1024
BATCH = 16384
SEQ = 200
DIM = 64

NC = 2
NS = 16
NW = NC * NS
LANES = 16
ROWS_PER_TILE = BATCH // NW
BLOCKS_PER_TILE = ROWS_PER_TILE // 128


def _proj_kernel(t_ref, wt_ref, b_ref, o_ref):
    o_ref[...] = (
        jnp.dot(t_ref[...], wt_ref[...], preferred_element_type=jnp.float32)
        + b_ref[...]
    )


def _sc_body(v_hbm, xt_hbm, out_hbm, v_v, xb_v, out_v):
    wid = lax.axis_index("s") * NC + lax.axis_index("c")

    pltpu.sync_copy(v_hbm, v_v)

    for ibloc in range(BLOCKS_PER_TILE):
        ib = wid * BLOCKS_PER_TILE + ibloc
        pltpu.sync_copy(xt_hbm.at[:, pl.ds(ib * 128, 128)], xb_v)

        def jhi_body(jhi, accs):
            accs = list(accs)
            for q in range(8):
                row = jhi * 8 + q
                for gl in range(8):
                    idx = xb_v[row, pl.ds(gl * 16, 16)]
                    vals = plsc.load_gather(v_v, [idx])
                    accs[gl] = accs[gl] + vals
            return tuple(accs)

        zero = jnp.zeros((16,), jnp.float32)
        accs = lax.fori_loop(0, SEQ // 8, jhi_body, (zero,) * 8)
        for gl in range(8):
            z = accs[gl] * (1.0 / SEQ)
            y = 1.0 / (1.0 + jnp.exp(-z))
            out_v[pl.ds(ibloc * 128 + gl * 16, 16)] = y

    pltpu.sync_copy(out_v, out_hbm.at[pl.ds(wid * ROWS_PER_TILE, ROWS_PER_TILE)])


_sc_call = functools.partial(
    pl.kernel,
    out_type=jax.ShapeDtypeStruct((BATCH,), jnp.float32),
    mesh=plsc.VectorSubcoreMesh(core_axis_name="c", subcore_axis_name="s"),
    scratch_types=[
        pltpu.VMEM((VPAD,), jnp.float32),
        pltpu.VMEM((SEQ, 128), jnp.int32),
        pltpu.VMEM((ROWS_PER_TILE,), jnp.float32),
    ],
    compiler_params=pltpu.CompilerParams(
        needs_layout_passes=False, use_tc_tiling_on_sc=True
    ),
)(_sc_body)


def kernel(x, emb_table, W, b):
    table_pad = jnp.pad(emb_table, ((0, VPAD - VOCAB), (0, 0)))
    wt = W.reshape(DIM, 1)
    b2 = b.reshape(1, 1)
    v = pl.pallas_call(
        _proj_kernel,
        out_shape=jax.ShapeDtypeStruct((VPAD, 1), jnp.float32),
    )(table_pad, wt, b2)
    out = _sc_call(v.reshape(VPAD), x.T)
    return out.reshape(BATCH, 1)


# speedup vs baseline: 2.4084x; 1.1530x over previous
import functools

import jax
import jax.numpy as jnp
from jax import lax
from jax.experimental import pallas as pl
from jax.experimental.pallas import tpu as pltpu
from jax.experimental.pallas import tpu_sc as plsc

VOCAB = 1000
VPAD = 1024
BATCH = 16384
SEQ = 200
DIM = 64

NC = 2
NS = 16
NW = NC * NS
LANES = 16
ROWS_PER_TILE = BATCH // NW
BLOCKS_PER_TILE = ROWS_PER_TILE // 128


def _proj_kernel(t_ref, wt_ref, b_ref, o_ref):
    o_ref[...] = (
        jnp.dot(t_ref[...], wt_ref[...], preferred_element_type=jnp.float32)
        + b_ref[...]
    )


def _sc_body(v_hbm, xt_hbm, out_hbm, v_v, xb0_v, xb1_v, out_v, sem0, sem1):
    wid = lax.axis_index("s") * NC + lax.axis_index("c")
    ib0 = wid * BLOCKS_PER_TILE

    xbufs = (xb0_v, xb1_v)
    sems = (sem0, sem1)

    def start(ibloc):
        return pltpu.async_copy(
            xt_hbm.at[:, pl.ds((ib0 + ibloc) * 128, 128)],
            xbufs[ibloc % 2],
            sems[ibloc % 2],
        )

    pending = start(0)
    pltpu.sync_copy(v_hbm, v_v)

    for ibloc in range(BLOCKS_PER_TILE):
        pending.wait()
        if ibloc + 1 < BLOCKS_PER_TILE:
            pending = start(ibloc + 1)
        xb_v = xbufs[ibloc % 2]

        def jhi_body(jhi, accs):
            accs = list(accs)
            for q in range(8):
                row = jhi * 8 + q
                for gl in range(8):
                    idx = xb_v[row, pl.ds(gl * 16, 16)]
                    vals = plsc.load_gather(v_v, [idx])
                    accs[gl] = accs[gl] + vals
            return tuple(accs)

        zero = jnp.zeros((16,), jnp.float32)
        accs = lax.fori_loop(0, SEQ // 8, jhi_body, (zero,) * 8)
        for gl in range(8):
            z = accs[gl] * (1.0 / SEQ)
            y = 1.0 / (1.0 + jnp.exp(-z))
            out_v[pl.ds(ibloc * 128 + gl * 16, 16)] = y

    pltpu.sync_copy(out_v, out_hbm.at[pl.ds(wid * ROWS_PER_TILE, ROWS_PER_TILE)])


_sc_call = functools.partial(
    pl.kernel,
    out_type=jax.ShapeDtypeStruct((BATCH,), jnp.float32),
    mesh=plsc.VectorSubcoreMesh(core_axis_name="c", subcore_axis_name="s"),
    scratch_types=[
        pltpu.VMEM((VPAD,), jnp.float32),
        pltpu.VMEM((SEQ, 128), jnp.int32),
        pltpu.VMEM((SEQ, 128), jnp.int32),
        pltpu.VMEM((ROWS_PER_TILE,), jnp.float32),
        pltpu.SemaphoreType.DMA,
        pltpu.SemaphoreType.DMA,
    ],
    compiler_params=pltpu.CompilerParams(
        needs_layout_passes=False, use_tc_tiling_on_sc=True
    ),
)(_sc_body)


def kernel(x, emb_table, W, b):
    table_pad = jnp.pad(emb_table, ((0, VPAD - VOCAB), (0, 0)))
    wt = W.reshape(DIM, 1)
    b2 = b.reshape(1, 1)
    v = pl.pallas_call(
        _proj_kernel,
        out_shape=jax.ShapeDtypeStruct((VPAD, 1), jnp.float32),
    )(table_pad, wt, b2)
    out = _sc_call(v.reshape(VPAD), x.T)
    return out.reshape(BATCH, 1)


# speedup vs baseline: 2.5582x; 1.0622x over previous
import functools

import jax
import jax.numpy as jnp
from jax import lax
from jax.experimental import pallas as pl
from jax.experimental.pallas import tpu as pltpu
from jax.experimental.pallas import tpu_sc as plsc

VOCAB = 1000
BATCH = 16384
SEQ = 200
DIM = 64

NC = 2
NS = 16
NW = NC * NS
LANES = 16
ROWS_PER_TILE = BATCH // NW
BLOCKS_PER_TILE = ROWS_PER_TILE // 128
VROWS = 64


def _sc_body(tf_hbm, wb_hbm, xt_hbm, out_hbm,
             v_v, xb0_v, xb1_v, out_v, tloc_v, wb_v, vloc_v, vsh_v,
             sem0, sem1):
    cid = lax.axis_index("c")
    sid = lax.axis_index("s")
    wid = sid * NC + cid
    ib0 = wid * BLOCKS_PER_TILE

    xbufs = (xb0_v, xb1_v)
    sems = (sem0, sem1)

    def start(ibloc):
        return pltpu.async_copy(
            xt_hbm.at[:, pl.ds((ib0 + ibloc) * 128, 128)],
            xbufs[ibloc % 2],
            sems[ibloc % 2],
        )

    pending = start(0)

    pltpu.sync_copy(wb_hbm, wb_v)
    row_start = jnp.minimum(sid * VROWS, VOCAB - VROWS)
    pltpu.sync_copy(tf_hbm.at[pl.ds(row_start * DIM, VROWS * DIM)], tloc_v)
    w0 = wb_v[pl.ds(0, 16)]
    w1 = wb_v[pl.ds(16, 16)]
    w2 = wb_v[pl.ds(32, 16)]
    w3 = wb_v[pl.ds(48, 16)]
    bval = wb_v[pl.ds(64, 16)][0]
    lane = lax.iota(jnp.int32, 16)
    vec = jnp.zeros((16,), jnp.float32)
    for r in range(VROWS):
        base = r * DIM
        acc = (tloc_v[pl.ds(base, 16)] * w0
               + tloc_v[pl.ds(base + 16, 16)] * w1
               + tloc_v[pl.ds(base + 32, 16)] * w2
               + tloc_v[pl.ds(base + 48, 16)] * w3)
        s = jnp.sum(acc)
        vec = jnp.where(lane == (r % 16), s, vec)
        if r % 16 == 15:
            vloc_v[pl.ds((r // 16) * 16, 16)] = vec
            vec = jnp.zeros((16,), jnp.float32)
    pltpu.sync_copy(vloc_v, vsh_v.at[pl.ds(row_start, VROWS)])
    plsc.subcore_barrier()
    pltpu.sync_copy(vsh_v, v_v)

    for ibloc in range(BLOCKS_PER_TILE):
        pending.wait()
        if ibloc + 1 < BLOCKS_PER_TILE:
            pending = start(ibloc + 1)
        xb_v = xbufs[ibloc % 2]

        def jhi_body(jhi, accs):
            accs = list(accs)
            for q in range(8):
                row = jhi * 8 + q
                for gl in range(8):
                    idx = xb_v[row, pl.ds(gl * 16, 16)]
                    vals = plsc.load_gather(v_v, [idx])
                    accs[gl] = accs[gl] + vals
            return tuple(accs)

        zero = jnp.zeros((16,), jnp.float32)
        accs = lax.fori_loop(0, SEQ // 8, jhi_body, (zero,) * 8)
        for gl in range(8):
            z = accs[gl] * (1.0 / SEQ) + bval
            y = 1.0 / (1.0 + jnp.exp(-z))
            out_v[pl.ds(ibloc * 128 + gl * 16, 16)] = y

    pltpu.sync_copy(out_v, out_hbm.at[pl.ds(wid * ROWS_PER_TILE, ROWS_PER_TILE)])


_sc_call = functools.partial(
    pl.kernel,
    out_type=jax.ShapeDtypeStruct((BATCH,), jnp.float32),
    mesh=plsc.VectorSubcoreMesh(core_axis_name="c", subcore_axis_name="s"),
    scratch_types=[
        pltpu.VMEM((VOCAB,), jnp.float32),
        pltpu.VMEM((SEQ, 128), jnp.int32),
        pltpu.VMEM((SEQ, 128), jnp.int32),
        pltpu.VMEM((ROWS_PER_TILE,), jnp.float32),
        pltpu.VMEM((VROWS * DIM,), jnp.float32),
        pltpu.VMEM((80,), jnp.float32),
        pltpu.VMEM((VROWS,), jnp.float32),
        pltpu.VMEM_SHARED((VOCAB,), jnp.float32),
        pltpu.SemaphoreType.DMA,
        pltpu.SemaphoreType.DMA,
    ],
    compiler_params=pltpu.CompilerParams(
        needs_layout_passes=False, use_tc_tiling_on_sc=True
    ),
)(_sc_body)


def kernel(x, emb_table, W, b):
    tflat = emb_table.reshape(VOCAB * DIM)
    wb = jnp.concatenate(
        [W.reshape(DIM), b, jnp.zeros(15, jnp.float32)]
    )
    out = _sc_call(tflat, wb, x.T)
    return out.reshape(BATCH, 1)
